# Initial kernel scaffold; baseline (speedup 1.0000x reference)
import jax
import jax.numpy as jnp
from jax.experimental import pallas as pl


def kernel(latents, embed_weight):
    raise NotImplementedError("write your pallas kernel here")


# speedup vs baseline: 12.4329x; 12.4329x over previous
import functools

import jax
import jax.numpy as jnp
from jax import lax
from jax.experimental import pallas as pl
from jax.experimental.pallas import tpu as pltpu
from jax.experimental.pallas import tpu_sc as plsc

K = 8192
D = 32
BETA = 0.25
N = 16384
BM = 256
NB = N // BM

_NC = 2
_NS = 16
_NW = _NC * _NS
_BPW = N // _NW


def _argmin_body(x_ref, w_ref, a_ref, b_ref, inds_ref, loss_ref):
    x = x_ref[...]
    w = w_ref[...]
    xb = x.astype(jnp.bfloat16)
    c = lax.dot_general(xb, w, (((1,), (1,)), ((), ())),
                        preferred_element_type=jnp.float32)
    a = a_ref[...]
    b = b_ref[...]
    dist = (a[:, None] + b[None, :]) - 2.0 * c
    m = jnp.min(dist, axis=1)
    iota = lax.broadcasted_iota(jnp.int32, (BM, K), 1)
    inds = jnp.min(jnp.where(dist == m[:, None], iota, K), axis=1)
    inds_ref[0, 0, :] = inds

    @pl.when(pl.program_id(0) == 0)
    def _():
        loss_ref[0, 0] = 0.0

    scale = (1.0 + BETA) / (N * D)
    loss_ref[0, 0] += jnp.sum(m) * scale


def _tc_argmin(flat, w, a, b):
    return pl.pallas_call(
        _argmin_body,
        grid=(NB,),
        in_specs=[
            pl.BlockSpec((BM, D), lambda i: (i, 0)),
            pl.BlockSpec((K, D), lambda i: (0, 0)),
            pl.BlockSpec((BM,), lambda i: (i,)),
            pl.BlockSpec((K,), lambda i: (0,)),
        ],
        out_specs=[
            pl.BlockSpec((1, 1, BM), lambda i: (i, 0, 0)),
            pl.BlockSpec(memory_space=pltpu.SMEM),
        ],
        out_shape=[
            jax.ShapeDtypeStruct((NB, 1, BM), jnp.int32),
            jax.ShapeDtypeStruct((1, 1), jnp.float32),
        ],
    )(flat, w, a, b)


@functools.cache
def _make_sc_gather():
    @functools.partial(
        pl.kernel,
        mesh=plsc.VectorSubcoreMesh(core_axis_name="c", subcore_axis_name="s"),
        out_type=jax.ShapeDtypeStruct((N, D), jnp.float32),
        scratch_types=[
            pltpu.VMEM((_BPW,), jnp.int32),
            pltpu.VMEM((_BPW, D), jnp.float32),
            pltpu.SemaphoreType.DMA,
        ],
        compiler_params=pltpu.CompilerParams(use_tc_tiling_on_sc=False),
    )
    def _sc_gather(table_hbm, idx_hbm, out_hbm, idx_v, rows_v, sem):
        wid = lax.axis_index("s") * _NC + lax.axis_index("c")
        base = wid * _BPW
        pltpu.sync_copy(idx_hbm.at[pl.ds(base, _BPW)], idx_v)
        pltpu.async_copy(table_hbm.at[idx_v], rows_v, sem).wait()
        pltpu.sync_copy(rows_v, out_hbm.at[pl.ds(base, _BPW)])

    return _sc_gather


def kernel(latents, embed_weight):
    shape = latents.shape
    flat = latents.reshape(-1, D)
    a = jnp.sum(latents ** 2, axis=-1).reshape(-1)
    b = jnp.sum(embed_weight ** 2, axis=1)
    inds3, loss = _tc_argmin(flat, embed_weight, a, b)
    inds = inds3.reshape(-1)
    quant = _make_sc_gather()(embed_weight, inds)
    quantized_st = quant.reshape(shape)
    inds_out = inds.reshape(shape[:-1] + (1,))
    return (quantized_st, loss[0, 0], inds_out)
